# baseline (speedup 1.0000x reference)
import functools

import jax
import jax.numpy as jnp
from jax.experimental import pallas as pl
from jax.experimental.pallas import tpu as pltpu

B, T, N = 2, 2, 131072
BEV_H = BEV_W = 512
RV_H, RV_W = 64, 2048
PN_CH = 64

PN_CHUNK = 4096
SC_CHUNK = 1024
SC_U = 8

NEG = float("-inf")


def _pointnet_body(x_ref, w1_ref, b1_ref, w2_ref, b2_ref, out_ref):
    x = x_ref[0]
    valid = (x[:, 4:5] < 100.0).astype(jnp.float32)
    x = x * valid
    h = jnp.maximum(
        jax.lax.dot_general(x, w1_ref[...], (((1,), (0,)), ((), ())),
                            preferred_element_type=jnp.float32) + b1_ref[...],
        0.0)
    f = jnp.maximum(
        jax.lax.dot_general(h, w2_ref[...], (((1,), (1,)), ((), ())),
                            preferred_element_type=jnp.float32) + b2_ref[...],
        0.0)
    f = f * valid
    out_ref[0, 0] = f[:, :32]
    out_ref[0, 1] = f[:, 32:]


def _pointnet(x8, w1, b1, w2, b2):
    bt, n, _ = x8.shape
    grid = (bt, n // PN_CHUNK)
    return pl.pallas_call(
        _pointnet_body,
        grid=grid,
        in_specs=[
            pl.BlockSpec((1, PN_CHUNK, 8), lambda i, j: (i, j, 0)),
            pl.BlockSpec((8, 64), lambda i, j: (0, 0)),
            pl.BlockSpec((1, 64), lambda i, j: (0, 0)),
            pl.BlockSpec((64, 64), lambda i, j: (0, 0)),
            pl.BlockSpec((1, 64), lambda i, j: (0, 0)),
        ],
        out_specs=pl.BlockSpec((1, 2, PN_CHUNK, 32), lambda i, j: (i, 0, j, 0)),
        out_shape=jax.ShapeDtypeStruct((bt, 2, n, 32), jnp.float32),
        compiler_params=pltpu.CompilerParams(
            dimension_semantics=("parallel", "arbitrary")),
    )(x8, w1, b1, w2, b2)


def _rmw_batch(acc_ref, pts, cs, iota_cell):
    pk = 128 // cs
    pksh = {4: 2, 2: 1, 1: 0}[pk]
    bases, grps, masked = [], [], []
    for idx, fv in pts:
        row = idx >> pksh
        base = pl.multiple_of((row >> 3) << 3, 8)
        fv128 = jnp.concatenate([fv] * pk) if pk > 1 else fv
        m = jnp.where(iota_cell == (idx & (8 * pk - 1)), fv128[None, :], NEG)
        bases.append(base)
        grps.append(row >> 3)
        masked.append(m)
    nu = len(pts)
    loaded = [acc_ref[pl.ds(bases[k], 8), :] for k in range(nu)]
    vals = []
    for k in range(nu):
        t = loaded[k]
        for j2 in range(k):
            t = jnp.where(grps[j2] == grps[k], vals[j2], t)
        vals.append(jnp.maximum(t, masked[k]))
    for k in range(nu):
        acc_ref[pl.ds(bases[k], 8), :] = vals[k]


def _acc_finalize(acc_ref, out_at, sem, nrow):
    strip = 4096

    def _clean(r, _):
        o = acc_ref[pl.ds(r * strip, strip), :]
        acc_ref[pl.ds(r * strip, strip), :] = jnp.where(
            jnp.isfinite(o), o, 0.0)
        return 0

    jax.lax.fori_loop(0, max(nrow // strip, 1), _clean, 0)
    cp = pltpu.make_async_copy(acc_ref, out_at, sem)
    cp.start()
    cp.wait()


def _acc_init(acc_ref, nrow):
    strip = 4096

    def _fill(r, _):
        acc_ref[pl.ds(r * strip, strip), :] = jnp.full(
            (strip, 128), NEG, jnp.float32)
        return 0

    jax.lax.fori_loop(0, max(nrow // strip, 1), _fill, 0)


def _linidx_body(r_ref, c_ref, o_ref, *, H, W, rscale, cscale):
    ri = jnp.clip(jnp.floor(r_ref[...] * rscale).astype(jnp.int32), 0, H - 1)
    ci = jnp.clip(jnp.floor(c_ref[...] * cscale).astype(jnp.int32), 0, W - 1)
    o_ref[...] = ri * W + ci


def _linidx(rows, cols, H, W, rscale=1.0, cscale=1.0):
    m = rows.shape[0]
    r2 = rows.reshape(m // 1024, 1024)
    c2 = cols.reshape(m // 1024, 1024)
    out = pl.pallas_call(
        functools.partial(_linidx_body, H=H, W=W, rscale=rscale, cscale=cscale),
        grid=(m // 8192,),
        in_specs=[pl.BlockSpec((8, 1024), lambda i: (i, 0)),
                  pl.BlockSpec((8, 1024), lambda i: (i, 0))],
        out_specs=pl.BlockSpec((8, 1024), lambda i: (i, 0)),
        out_shape=jax.ShapeDtypeStruct((m // 1024, 1024), jnp.int32),
    )(r2, c2)
    return out.reshape(m)


def _scatter_body(idx_ref, feat_ref, out_ref, acc_ref, sem, *, hw, cs, nch):
    bi = pl.program_id(0)
    hi = pl.program_id(1)
    j = pl.program_id(2)
    nrow = hw // (128 // cs)

    @pl.when(j == 0)
    def _init():
        _acc_init(acc_ref, nrow)

    iota_cell = (jax.lax.broadcasted_iota(jnp.int32, (8, 128), 0) * (128 // cs)
                 + jax.lax.broadcasted_iota(jnp.int32, (8, 128), 1) // cs)
    for u0 in range(0, SC_CHUNK, SC_U):
        pts = [(idx_ref[0, 0, 0, u0 + k], feat_ref[0, 0, u0 + k, :])
               for k in range(SC_U)]
        _rmw_batch(acc_ref, pts, cs, iota_cell)

    @pl.when(j == nch - 1)
    def _fin():
        _acc_finalize(acc_ref, out_ref.at[bi, hi], sem, nrow)


def _scatter_max_p(feat, idx, hw, feat_map):
    g_, nh, n_, cs = feat.shape
    b_, m = idx.shape
    nch = m // SC_CHUNK
    idx4 = idx.reshape(b_, nch, 1, SC_CHUNK)
    body = functools.partial(_scatter_body, hw=hw, cs=cs, nch=nch)
    return pl.pallas_call(
        body,
        grid=(b_, nh, nch),
        in_specs=[
            pl.BlockSpec((1, 1, 1, SC_CHUNK), lambda b, h, j: (b, j, 0, 0),
                         memory_space=pltpu.SMEM),
            pl.BlockSpec((1, 1, SC_CHUNK, cs), feat_map),
        ],
        out_specs=pl.BlockSpec(memory_space=pl.ANY),
        out_shape=jax.ShapeDtypeStruct((b_, nh, hw // (128 // cs), 128),
                                       jnp.float32),
        scratch_shapes=[pltpu.VMEM((hw // (128 // cs), 128), jnp.float32),
                        pltpu.SemaphoreType.DMA],
        compiler_params=pltpu.CompilerParams(
            dimension_semantics=("parallel", "arbitrary", "arbitrary")),
    )(idx4, feat).reshape(b_, nh, hw, cs)


def _bilidx_body(r_ref, c_ref, i00_ref, dr_ref, dc_ref, fr_ref, fc_ref,
                 *, H, W, scale):
    r = r_ref[...] * scale
    c = c_ref[...] * scale
    rf = jnp.floor(r)
    cf = jnp.floor(c)
    fr_ref[...] = r - rf
    fc_ref[...] = c - cf
    r0 = jnp.clip(rf.astype(jnp.int32), 0, H - 1)
    r1 = jnp.clip(r0 + 1, 0, H - 1)
    c0 = jnp.clip(cf.astype(jnp.int32), 0, W - 1)
    c1 = jnp.clip(c0 + 1, 0, W - 1)
    i00_ref[...] = r0 * W + c0
    dr_ref[...] = (r1 - r0) * W
    dc_ref[...] = c1 - c0


def _bilidx(rows, cols, H, W, scale):
    m = rows.shape[0]
    r2 = rows.reshape(m // 1024, 1024)
    c2 = cols.reshape(m // 1024, 1024)
    sp = pl.BlockSpec((8, 1024), lambda i: (i, 0))
    outs = pl.pallas_call(
        functools.partial(_bilidx_body, H=H, W=W, scale=scale),
        grid=(m // 8192,),
        in_specs=[sp, sp],
        out_specs=[sp] * 5,
        out_shape=[jax.ShapeDtypeStruct((m // 1024, 1024), jnp.int32)] * 3
        + [jax.ShapeDtypeStruct((m // 1024, 1024), jnp.float32)] * 2,
    )(r2, c2)
    return [o.reshape(m) for o in outs]


def _gather_bilin(src_ref, i00, dr, dc, fr, fc):
    v00 = src_ref[i00, 0, :]
    v01 = src_ref[i00 + dc, 0, :]
    v10 = src_ref[i00 + dr, 0, :]
    v11 = src_ref[i00 + dr + dc, 0, :]
    top = v00 + fc * (v01 - v00)
    bot = v10 + fc * (v11 - v10)
    return top + fr * (bot - top)


SH_HW = 256 * 256
DP_HW = 128 * 128


def _rvproj_body(sidx_ref, didx_ref,
                 si_ref, sdr_ref, sdc_ref, sfr_ref, sfc_ref,
                 di_ref, ddr_ref, ddc_ref, dfr_ref, dfc_ref,
                 sh_hbm, dp_hbm, sh_out, dp_out,
                 sh_src, dp_src, acc_sh, acc_dp, sem1, sem2, *, nch):
    bi = pl.program_id(0)
    j = pl.program_id(1)

    @pl.when(j == 0)
    def _load():
        cp1 = pltpu.make_async_copy(sh_hbm.at[bi], sh_src, sem1)
        cp2 = pltpu.make_async_copy(dp_hbm.at[bi], dp_src, sem2)
        cp1.start()
        cp2.start()
        cp1.wait()
        cp2.wait()
        _acc_init(acc_sh, SH_HW // 4)
        _acc_init(acc_dp, DP_HW // 4)

    iota_cell = (jax.lax.broadcasted_iota(jnp.int32, (8, 128), 0) * 4
                 + jax.lax.broadcasted_iota(jnp.int32, (8, 128), 1) // 32)
    for u0 in range(0, SC_CHUNK, SC_U):
        sh_pts, dp_pts = [], []
        for k in range(SC_U):
            i = u0 + k
            sv = _gather_bilin(sh_src, si_ref[0, 0, 0, i], sdr_ref[0, 0, 0, i],
                               sdc_ref[0, 0, 0, i], sfr_ref[0, 0, 0, i],
                               sfc_ref[0, 0, 0, i])
            dv = _gather_bilin(dp_src, di_ref[0, 0, 0, i], ddr_ref[0, 0, 0, i],
                               ddc_ref[0, 0, 0, i], dfr_ref[0, 0, 0, i],
                               dfc_ref[0, 0, 0, i])
            sh_pts.append((sidx_ref[0, 0, 0, i], sv))
            dp_pts.append((didx_ref[0, 0, 0, i], dv))
        _rmw_batch(acc_sh, sh_pts, 32, iota_cell)
        _rmw_batch(acc_dp, dp_pts, 32, iota_cell)

    @pl.when(j == nch - 1)
    def _fin():
        _acc_finalize(acc_sh, sh_out.at[bi], sem1, SH_HW // 4)
        _acc_finalize(acc_dp, dp_out.at[bi], sem2, DP_HW // 4)


def _rvproj(shallow_hwc, deep_hwc, sidx, didx, sbil, dbil):
    b_ = shallow_hwc.shape[0]
    n_ = sidx.shape[1]
    nch = n_ // SC_CHUNK
    sm = lambda b, j: (b, j, 0, 0)
    smem = functools.partial(pl.BlockSpec, (1, 1, 1, SC_CHUNK),
                             memory_space=pltpu.SMEM)
    rs = lambda a: a.reshape(b_, nch, 1, SC_CHUNK)
    ins = ([rs(sidx), rs(didx)] + [rs(a) for a in sbil] + [rs(a) for a in dbil]
           + [shallow_hwc, deep_hwc])
    sh_hw4 = SH_HW // 4
    dp_hw4 = DP_HW // 4
    sh_out, dp_out = pl.pallas_call(
        functools.partial(_rvproj_body, nch=nch),
        grid=(b_, nch),
        in_specs=[smem(sm)] * 12 + [pl.BlockSpec(memory_space=pl.ANY)] * 2,
        out_specs=[pl.BlockSpec(memory_space=pl.ANY)] * 2,
        out_shape=[jax.ShapeDtypeStruct((b_, sh_hw4, 128), jnp.float32),
                   jax.ShapeDtypeStruct((b_, dp_hw4, 128), jnp.float32)],
        scratch_shapes=[
            pltpu.VMEM((32 * 1024, 1, 32), jnp.float32),
            pltpu.VMEM((16 * 512, 1, 32), jnp.float32),
            pltpu.VMEM((sh_hw4, 128), jnp.float32),
            pltpu.VMEM((dp_hw4, 128), jnp.float32),
            pltpu.SemaphoreType.DMA, pltpu.SemaphoreType.DMA,
        ],
        compiler_params=pltpu.CompilerParams(
            dimension_semantics=("parallel", "arbitrary")),
    )(*ins)
    return (sh_out.reshape(b_, 1, SH_HW, 32), dp_out.reshape(b_, 1, DP_HW, 32))


MV_HW = 256 * 256


def _fuse_body(mi_ref, mdr_ref, mdc_ref, mfr_ref, mfc_ref,
               f0a_ref, f0b_ref, w_ref, bias_ref, mv_hbm, out_ref,
               mv_src, gsl, sem, *, nch):
    bi = pl.program_id(0)
    j = pl.program_id(1)

    @pl.when(j == 0)
    def _load():
        cp = pltpu.make_async_copy(mv_hbm.at[bi], mv_src, sem)
        cp.start()
        cp.wait()

    for i in range(SC_CHUNK):
        gsl[i, :] = _gather_bilin(mv_src, mi_ref[0, 0, 0, i],
                                  mdr_ref[0, 0, 0, i], mdc_ref[0, 0, 0, i],
                                  mfr_ref[0, 0, 0, i], mfc_ref[0, 0, 0, i])
    dn = (((1,), (0,)), ((), ()))
    logits = (
        jax.lax.dot_general(f0a_ref[0, 0], w_ref[0:32], dn,
                            preferred_element_type=jnp.float32)
        + jax.lax.dot_general(f0b_ref[0, 0], w_ref[32:64], dn,
                              preferred_element_type=jnp.float32)
        + jax.lax.dot_general(gsl[...], w_ref[64:96], dn,
                              preferred_element_type=jnp.float32))
    out_ref[0] = logits + bias_ref[...]


def _fuse(f_half, mv_hwc, mbil, fuse_w, fuse_b, t):
    bt, _, n_, _ = f_half.shape
    b_ = bt // t
    nch = n_ // SC_CHUNK
    smem = functools.partial(pl.BlockSpec, (1, 1, 1, SC_CHUNK),
                             memory_space=pltpu.SMEM)
    sm = lambda b, j: (b, j, 0, 0)
    rs = lambda a: a.reshape(b_, nch, 1, SC_CHUNK)
    return pl.pallas_call(
        functools.partial(_fuse_body, nch=nch),
        grid=(b_, nch),
        in_specs=[smem(sm)] * 5 + [
            pl.BlockSpec((1, 1, SC_CHUNK, 32),
                         lambda b, j: (t * b + t - 1, 0, j, 0)),
            pl.BlockSpec((1, 1, SC_CHUNK, 32),
                         lambda b, j: (t * b + t - 1, 1, j, 0)),
            pl.BlockSpec((96, 3), lambda b, j: (0, 0)),
            pl.BlockSpec((1, 3), lambda b, j: (0, 0)),
            pl.BlockSpec(memory_space=pl.ANY),
        ],
        out_specs=pl.BlockSpec((1, SC_CHUNK, 3), lambda b, j: (b, j, 0)),
        out_shape=jax.ShapeDtypeStruct((b_, n_, 3), jnp.float32),
        scratch_shapes=[
            pltpu.VMEM((MV_HW, 1, 32), jnp.float32),
            pltpu.VMEM((SC_CHUNK, 32), jnp.float32),
            pltpu.SemaphoreType.DMA,
        ],
        compiler_params=pltpu.CompilerParams(
            dimension_semantics=("parallel", "arbitrary")),
    )(*([rs(a) for a in mbil] + [f_half, f_half, fuse_w, fuse_b, mv_hwc]))


def _conv_x(x, w, b, stride=1):
    y = jax.lax.conv_general_dilated(x, w, (stride, stride), 'SAME',
                                     dimension_numbers=('NCHW', 'OIHW', 'NCHW'))
    return y + b[None, :, None, None]


def _up2x(x):
    return jnp.repeat(jnp.repeat(x, 2, axis=2), 2, axis=3)


def _smax(feat, rows, cols, bidx, nb, H, W):
    ri = jnp.clip(jnp.floor(rows).astype(jnp.int32), 0, H - 1)
    ci = jnp.clip(jnp.floor(cols).astype(jnp.int32), 0, W - 1)
    idx = (bidx * H + ri) * W + ci
    g = jax.ops.segment_max(feat, idx, num_segments=nb * H * W)
    g = jnp.where(jnp.isfinite(g), g, 0.0)
    return g.reshape(nb, H, W, -1).transpose(0, 3, 1, 2)


def _bilin(fmap, coords, scale):
    H, W = fmap.shape[2], fmap.shape[3]

    def one(f, rc):
        r = rc[:, 0] * scale
        c = rc[:, 1] * scale
        r0 = jnp.floor(r)
        c0 = jnp.floor(c)
        fr = r - r0
        fc = c - c0
        r0i = jnp.clip(r0.astype(jnp.int32), 0, H - 1)
        r1i = jnp.clip(r0i + 1, 0, H - 1)
        c0i = jnp.clip(c0.astype(jnp.int32), 0, W - 1)
        c1i = jnp.clip(c0i + 1, 0, W - 1)
        return (f[:, r0i, c0i] * (1 - fr) * (1 - fc)
                + f[:, r0i, c1i] * (1 - fr) * fc
                + f[:, r1i, c0i] * fr * (1 - fc)
                + f[:, r1i, c1i] * fr * fc)

    return jax.vmap(one)(fmap, coords)


def kernel(xyzi, des_coord, sph_coord, params):
    p = params
    b, t = B, T
    n = N

    x = xyzi[..., 0].reshape(b * t, 7, n).transpose(0, 2, 1)
    x8 = jnp.pad(x, ((0, 0), (0, 0), (0, 1)))
    w1 = jnp.pad(p['pn_w1'][:, :, 0, 0], ((0, 0), (0, 1))).T
    w2 = p['pn_w2'][:, :, 0, 0]
    f_half = _pointnet(x8, w1, p['pn_b1'][None, :], w2, p['pn_b2'][None, :])
    nchf = n // SC_CHUNK

    bev_rows = des_coord[:, :, :, 0, 0].reshape(-1)
    bev_cols = des_coord[:, :, :, 1, 0].reshape(-1)
    bev_idx = _linidx(bev_rows, bev_cols, BEV_H, BEV_W).reshape(b, t * n)
    bev_flat = _scatter_max_p(
        f_half, bev_idx, BEV_H * BEV_W,
        lambda bb, h, j: (t * bb + j // nchf, h, j % nchf, 0))
    bev_feat = (bev_flat.reshape(b, 2, BEV_H, BEV_W, 32)
                .transpose(0, 1, 4, 2, 3).reshape(b, PN_CH, BEV_H, BEV_W))

    rv_rows = sph_coord[:, -1, :, 1, 0].reshape(-1)
    rv_cols = sph_coord[:, -1, :, 0, 0].reshape(-1)
    rv_idx = _linidx(rv_rows, rv_cols, RV_H, RV_W).reshape(b, n)
    rv_flat = _scatter_max_p(
        f_half, rv_idx, RV_H * RV_W,
        lambda bb, h, j: (t * bb + (t - 1), h, j, 0))
    rv_feat = (rv_flat.reshape(b, 2, RV_H, RV_W, 32)
               .transpose(0, 1, 4, 2, 3).reshape(b, PN_CH, RV_H, RV_W))

    shallow = jax.nn.relu(_conv_x(rv_feat, p['rv_c1_w'], p['rv_c1_b'], 2))
    deep = jax.nn.relu(_conv_x(shallow, p['rv_c2_w'], p['rv_c2_b'], 2))
    movable_logit_2d = _up2x(_conv_x(shallow, p['rv_head_w'], p['rv_head_b']))

    rvr = sph_coord[:, -1, :, 1, 0].reshape(-1)
    rvc = sph_coord[:, -1, :, 0, 0].reshape(-1)
    sbil = _bilidx(rvr, rvc, RV_H // 2, RV_W // 2, 0.5)
    dbil = _bilidx(rvr, rvc, RV_H // 4, RV_W // 4, 0.25)

    bev_r0 = des_coord[:, -1, :, 0, 0]
    bev_c0 = des_coord[:, -1, :, 1, 0]
    sh_idx = _linidx((bev_r0 * 0.5).reshape(-1), (bev_c0 * 0.5).reshape(-1),
                     BEV_H // 2, BEV_W // 2).reshape(b, n)
    dp_idx = _linidx((bev_r0 * 0.25).reshape(-1), (bev_c0 * 0.25).reshape(-1),
                     BEV_H // 4, BEV_W // 4).reshape(b, n)

    sh_hwc = shallow.transpose(0, 2, 3, 1).reshape(b, RV_H // 2 * (RV_W // 2), 1, 32)
    dp_hwc = deep.transpose(0, 2, 3, 1).reshape(b, RV_H // 4 * (RV_W // 4), 1, 32)
    sh_flat, dp_flat = _rvproj(sh_hwc, dp_hwc, sh_idx, dp_idx, sbil, dbil)
    sh_bev = sh_flat.reshape(b, BEV_H // 2, BEV_W // 2, 32).transpose(0, 3, 1, 2)
    dp_bev = dp_flat.reshape(b, BEV_H // 4, BEV_W // 4, 32).transpose(0, 3, 1, 2)

    x1 = jax.nn.relu(_conv_x(bev_feat, p['bev_c1_w'], p['bev_c1_b'], 2))
    x1 = jax.nn.relu(_conv_x(jnp.concatenate([x1, sh_bev], 1), p['bev_c2_w'], p['bev_c2_b']))
    x2 = jax.nn.relu(_conv_x(x1, p['bev_c3_w'], p['bev_c3_b'], 2))
    x2 = jax.nn.relu(_conv_x(jnp.concatenate([x2, dp_bev], 1), p['bev_c4_w'], p['bev_c4_b']))
    moving_feat_2d = _conv_x(x1 + _up2x(x2), p['bev_c5_w'], p['bev_c5_b'])

    mbil = _bilidx(bev_r0.reshape(-1), bev_c0.reshape(-1),
                   BEV_H // 2, BEV_W // 2, 0.5)
    mv_hwc = moving_feat_2d.transpose(0, 2, 3, 1).reshape(b, MV_HW, 1, 32)
    logits = _fuse(f_half, mv_hwc, mbil, p['fuse_w'][:, :, 0, 0].T,
                   p['fuse_b'][None, :], t)
    moving_logit_3d = logits.transpose(0, 2, 1)[..., None]
    return moving_logit_3d, movable_logit_2d


# speedup vs baseline: 1.3197x; 1.3197x over previous
import functools

import jax
import jax.numpy as jnp
from jax.experimental import pallas as pl
from jax.experimental.pallas import tpu as pltpu

B, T, N = 2, 2, 131072
BEV_H = BEV_W = 512
RV_H, RV_W = 64, 2048
PN_CH = 64

PN_CHUNK = 4096
SC_CHUNK = 512
SC_U = 4

NEG = float("-inf")


def _pointnet_body(x_ref, w1_ref, b1_ref, w2_ref, b2_ref, out_ref):
    x = x_ref[0]
    valid = (x[:, 4:5] < 100.0).astype(jnp.float32)
    x = x * valid
    h = jnp.maximum(
        jax.lax.dot_general(x, w1_ref[...], (((1,), (0,)), ((), ())),
                            preferred_element_type=jnp.float32) + b1_ref[...],
        0.0)
    f = jnp.maximum(
        jax.lax.dot_general(h, w2_ref[...], (((1,), (1,)), ((), ())),
                            preferred_element_type=jnp.float32) + b2_ref[...],
        0.0)
    f = f * valid
    out_ref[0, 0] = f[:, :32]
    out_ref[0, 1] = f[:, 32:]


def _pointnet(x8, w1, b1, w2, b2):
    bt, n, _ = x8.shape
    grid = (bt, n // PN_CHUNK)
    return pl.pallas_call(
        _pointnet_body,
        grid=grid,
        in_specs=[
            pl.BlockSpec((1, PN_CHUNK, 8), lambda i, j: (i, j, 0)),
            pl.BlockSpec((8, 64), lambda i, j: (0, 0)),
            pl.BlockSpec((1, 64), lambda i, j: (0, 0)),
            pl.BlockSpec((64, 64), lambda i, j: (0, 0)),
            pl.BlockSpec((1, 64), lambda i, j: (0, 0)),
        ],
        out_specs=pl.BlockSpec((1, 2, PN_CHUNK, 32), lambda i, j: (i, 0, j, 0)),
        out_shape=jax.ShapeDtypeStruct((bt, 2, n, 32), jnp.float32),
        compiler_params=pltpu.CompilerParams(
            dimension_semantics=("parallel", "arbitrary")),
    )(x8, w1, b1, w2, b2)


def _rmw_batch(acc_ref, pts, cs, iota_cell):
    pk = 128 // cs
    pksh = {4: 2, 2: 1, 1: 0}[pk]
    bases, grps, masked = [], [], []
    for idx, fv in pts:
        row = idx >> pksh
        base = pl.multiple_of((row >> 3) << 3, 8)
        fv128 = jnp.concatenate([fv] * pk) if pk > 1 else fv
        m = jnp.where(iota_cell == (idx & (8 * pk - 1)), fv128[None, :], NEG)
        bases.append(base)
        grps.append(row >> 3)
        masked.append(m)
    nu = len(pts)
    loaded = [acc_ref[pl.ds(bases[k], 8), :] for k in range(nu)]
    vals = []
    for k in range(nu):
        t = loaded[k]
        for j2 in range(k):
            t = jnp.where(grps[j2] == grps[k], vals[j2], t)
        vals.append(jnp.maximum(t, masked[k]))
    for k in range(nu):
        acc_ref[pl.ds(bases[k], 8), :] = vals[k]


def _acc_finalize(acc_ref, out_at, sem, nrow):
    strip = 4096

    def _clean(r, _):
        o = acc_ref[pl.ds(r * strip, strip), :]
        acc_ref[pl.ds(r * strip, strip), :] = jnp.where(
            jnp.isfinite(o), o, 0.0)
        return 0

    jax.lax.fori_loop(0, max(nrow // strip, 1), _clean, 0)
    cp = pltpu.make_async_copy(acc_ref, out_at, sem)
    cp.start()
    cp.wait()


def _acc_init(acc_ref, nrow):
    strip = 4096

    def _fill(r, _):
        acc_ref[pl.ds(r * strip, strip), :] = jnp.full(
            (strip, 128), NEG, jnp.float32)
        return 0

    jax.lax.fori_loop(0, max(nrow // strip, 1), _fill, 0)


def _linidx_body(r_ref, c_ref, o_ref, *, H, W, rscale, cscale):
    ri = jnp.clip(jnp.floor(r_ref[...] * rscale).astype(jnp.int32), 0, H - 1)
    ci = jnp.clip(jnp.floor(c_ref[...] * cscale).astype(jnp.int32), 0, W - 1)
    o_ref[...] = ri * W + ci


def _linidx(rows, cols, H, W, rscale=1.0, cscale=1.0):
    m = rows.shape[0]
    r2 = rows.reshape(m // 1024, 1024)
    c2 = cols.reshape(m // 1024, 1024)
    out = pl.pallas_call(
        functools.partial(_linidx_body, H=H, W=W, rscale=rscale, cscale=cscale),
        grid=(m // 8192,),
        in_specs=[pl.BlockSpec((8, 1024), lambda i: (i, 0)),
                  pl.BlockSpec((8, 1024), lambda i: (i, 0))],
        out_specs=pl.BlockSpec((8, 1024), lambda i: (i, 0)),
        out_shape=jax.ShapeDtypeStruct((m // 1024, 1024), jnp.int32),
    )(r2, c2)
    return out.reshape(m)


def _scatter_body(idx_ref, feat_ref, out_ref, acc_ref, sem, *, hw, cs, nch):
    bi = pl.program_id(0)
    hi = pl.program_id(1)
    j = pl.program_id(2)
    nrow = hw // (128 // cs)

    @pl.when(j == 0)
    def _init():
        _acc_init(acc_ref, nrow)

    iota_cell = (jax.lax.broadcasted_iota(jnp.int32, (8, 128), 0) * (128 // cs)
                 + jax.lax.broadcasted_iota(jnp.int32, (8, 128), 1) // cs)
    for u0 in range(0, SC_CHUNK, SC_U):
        pts = [(idx_ref[0, 0, 0, u0 + k], feat_ref[0, 0, u0 + k, :])
               for k in range(SC_U)]
        _rmw_batch(acc_ref, pts, cs, iota_cell)

    @pl.when(j == nch - 1)
    def _fin():
        _acc_finalize(acc_ref, out_ref.at[bi, hi], sem, nrow)


def _scatter_max_p(feat, idx, hw, feat_map):
    g_, nh, n_, cs = feat.shape
    b_, m = idx.shape
    nch = m // SC_CHUNK
    idx4 = idx.reshape(b_, nch, 1, SC_CHUNK)
    body = functools.partial(_scatter_body, hw=hw, cs=cs, nch=nch)
    return pl.pallas_call(
        body,
        grid=(b_, nh, nch),
        in_specs=[
            pl.BlockSpec((1, 1, 1, SC_CHUNK), lambda b, h, j: (b, j, 0, 0),
                         memory_space=pltpu.SMEM),
            pl.BlockSpec((1, 1, SC_CHUNK, cs), feat_map),
        ],
        out_specs=pl.BlockSpec(memory_space=pl.ANY),
        out_shape=jax.ShapeDtypeStruct((b_, nh, hw // (128 // cs), 128),
                                       jnp.float32),
        scratch_shapes=[pltpu.VMEM((hw // (128 // cs), 128), jnp.float32),
                        pltpu.SemaphoreType.DMA],
        compiler_params=pltpu.CompilerParams(
            dimension_semantics=("parallel", "arbitrary", "arbitrary")),
    )(idx4, feat).reshape(b_, nh, hw, cs)


def _bilidx_body(r_ref, c_ref, i00_ref, dr_ref, dc_ref, fr_ref, fc_ref,
                 *, H, W, scale):
    r = r_ref[...] * scale
    c = c_ref[...] * scale
    rf = jnp.floor(r)
    cf = jnp.floor(c)
    fr_ref[...] = r - rf
    fc_ref[...] = c - cf
    r0 = jnp.clip(rf.astype(jnp.int32), 0, H - 1)
    r1 = jnp.clip(r0 + 1, 0, H - 1)
    c0 = jnp.clip(cf.astype(jnp.int32), 0, W - 1)
    c1 = jnp.clip(c0 + 1, 0, W - 1)
    i00_ref[...] = r0 * W + c0
    dr_ref[...] = (r1 - r0) * W
    dc_ref[...] = c1 - c0


def _bilidx(rows, cols, H, W, scale):
    m = rows.shape[0]
    r2 = rows.reshape(m // 1024, 1024)
    c2 = cols.reshape(m // 1024, 1024)
    sp = pl.BlockSpec((8, 1024), lambda i: (i, 0))
    outs = pl.pallas_call(
        functools.partial(_bilidx_body, H=H, W=W, scale=scale),
        grid=(m // 8192,),
        in_specs=[sp, sp],
        out_specs=[sp] * 5,
        out_shape=[jax.ShapeDtypeStruct((m // 1024, 1024), jnp.int32)] * 3
        + [jax.ShapeDtypeStruct((m // 1024, 1024), jnp.float32)] * 2,
    )(r2, c2)
    return [o.reshape(m) for o in outs]


def _gather_bilin(src_ref, i00, dr, dc, fr, fc):
    v00 = src_ref[i00, 0, :]
    v01 = src_ref[i00 + dc, 0, :]
    v10 = src_ref[i00 + dr, 0, :]
    v11 = src_ref[i00 + dr + dc, 0, :]
    top = v00 + fc * (v01 - v00)
    bot = v10 + fc * (v11 - v10)
    return top + fr * (bot - top)


SH_HW = 256 * 256
DP_HW = 128 * 128


def _rvproj_body(sidx_ref, didx_ref,
                 si_ref, sdr_ref, sdc_ref, sfr_ref, sfc_ref,
                 di_ref, ddr_ref, ddc_ref, dfr_ref, dfc_ref,
                 sh_hbm, dp_hbm, sh_out, dp_out,
                 sh_src, dp_src, acc_sh, acc_dp, sem1, sem2, *, nch):
    bi = pl.program_id(0)
    j = pl.program_id(1)

    @pl.when(j == 0)
    def _load():
        cp1 = pltpu.make_async_copy(sh_hbm.at[bi], sh_src, sem1)
        cp2 = pltpu.make_async_copy(dp_hbm.at[bi], dp_src, sem2)
        cp1.start()
        cp2.start()
        cp1.wait()
        cp2.wait()
        _acc_init(acc_sh, SH_HW // 4)
        _acc_init(acc_dp, DP_HW // 4)

    iota_cell = (jax.lax.broadcasted_iota(jnp.int32, (8, 128), 0) * 4
                 + jax.lax.broadcasted_iota(jnp.int32, (8, 128), 1) // 32)
    for u0 in range(0, SC_CHUNK, SC_U):
        sh_pts, dp_pts = [], []
        for k in range(SC_U):
            i = u0 + k
            sv = _gather_bilin(sh_src, si_ref[0, 0, 0, i], sdr_ref[0, 0, 0, i],
                               sdc_ref[0, 0, 0, i], sfr_ref[0, 0, 0, i],
                               sfc_ref[0, 0, 0, i])
            dv = _gather_bilin(dp_src, di_ref[0, 0, 0, i], ddr_ref[0, 0, 0, i],
                               ddc_ref[0, 0, 0, i], dfr_ref[0, 0, 0, i],
                               dfc_ref[0, 0, 0, i])
            sh_pts.append((sidx_ref[0, 0, 0, i], sv))
            dp_pts.append((didx_ref[0, 0, 0, i], dv))
        _rmw_batch(acc_sh, sh_pts, 32, iota_cell)
        _rmw_batch(acc_dp, dp_pts, 32, iota_cell)

    @pl.when(j == nch - 1)
    def _fin():
        _acc_finalize(acc_sh, sh_out.at[bi], sem1, SH_HW // 4)
        _acc_finalize(acc_dp, dp_out.at[bi], sem2, DP_HW // 4)


def _rvproj(shallow_hwc, deep_hwc, sidx, didx, sbil, dbil):
    b_ = shallow_hwc.shape[0]
    n_ = sidx.shape[1]
    nch = n_ // SC_CHUNK
    sm = lambda b, j: (b, j, 0, 0)
    smem = functools.partial(pl.BlockSpec, (1, 1, 1, SC_CHUNK),
                             memory_space=pltpu.SMEM)
    rs = lambda a: a.reshape(b_, nch, 1, SC_CHUNK)
    ins = ([rs(sidx), rs(didx)] + [rs(a) for a in sbil] + [rs(a) for a in dbil]
           + [shallow_hwc, deep_hwc])
    sh_hw4 = SH_HW // 4
    dp_hw4 = DP_HW // 4
    sh_out, dp_out = pl.pallas_call(
        functools.partial(_rvproj_body, nch=nch),
        grid=(b_, nch),
        in_specs=[smem(sm)] * 12 + [pl.BlockSpec(memory_space=pl.ANY)] * 2,
        out_specs=[pl.BlockSpec(memory_space=pl.ANY)] * 2,
        out_shape=[jax.ShapeDtypeStruct((b_, sh_hw4, 128), jnp.float32),
                   jax.ShapeDtypeStruct((b_, dp_hw4, 128), jnp.float32)],
        scratch_shapes=[
            pltpu.VMEM((32 * 1024, 1, 32), jnp.float32),
            pltpu.VMEM((16 * 512, 1, 32), jnp.float32),
            pltpu.VMEM((sh_hw4, 128), jnp.float32),
            pltpu.VMEM((dp_hw4, 128), jnp.float32),
            pltpu.SemaphoreType.DMA, pltpu.SemaphoreType.DMA,
        ],
        compiler_params=pltpu.CompilerParams(
            dimension_semantics=("parallel", "arbitrary")),
    )(*ins)
    return (sh_out.reshape(b_, 1, SH_HW, 32), dp_out.reshape(b_, 1, DP_HW, 32))


MV_HW = 256 * 256


def _fuse_body(mi_ref, mdr_ref, mdc_ref, mfr_ref, mfc_ref,
               f0a_ref, f0b_ref, w_ref, bias_ref, mv_hbm, out_ref,
               mv_src, gsl, sem, *, nch):
    bi = pl.program_id(0)
    j = pl.program_id(1)

    @pl.when(j == 0)
    def _load():
        cp = pltpu.make_async_copy(mv_hbm.at[bi], mv_src, sem)
        cp.start()
        cp.wait()

    for i in range(SC_CHUNK):
        gsl[i, :] = _gather_bilin(mv_src, mi_ref[0, 0, 0, i],
                                  mdr_ref[0, 0, 0, i], mdc_ref[0, 0, 0, i],
                                  mfr_ref[0, 0, 0, i], mfc_ref[0, 0, 0, i])
    dn = (((1,), (0,)), ((), ()))
    logits = (
        jax.lax.dot_general(f0a_ref[0, 0], w_ref[0:32], dn,
                            preferred_element_type=jnp.float32)
        + jax.lax.dot_general(f0b_ref[0, 0], w_ref[32:64], dn,
                              preferred_element_type=jnp.float32)
        + jax.lax.dot_general(gsl[...], w_ref[64:96], dn,
                              preferred_element_type=jnp.float32))
    out_ref[0] = logits + bias_ref[...]


def _fuse(f_half, mv_hwc, mbil, fuse_w, fuse_b, t):
    bt, _, n_, _ = f_half.shape
    b_ = bt // t
    nch = n_ // SC_CHUNK
    smem = functools.partial(pl.BlockSpec, (1, 1, 1, SC_CHUNK),
                             memory_space=pltpu.SMEM)
    sm = lambda b, j: (b, j, 0, 0)
    rs = lambda a: a.reshape(b_, nch, 1, SC_CHUNK)
    return pl.pallas_call(
        functools.partial(_fuse_body, nch=nch),
        grid=(b_, nch),
        in_specs=[smem(sm)] * 5 + [
            pl.BlockSpec((1, 1, SC_CHUNK, 32),
                         lambda b, j: (t * b + t - 1, 0, j, 0)),
            pl.BlockSpec((1, 1, SC_CHUNK, 32),
                         lambda b, j: (t * b + t - 1, 1, j, 0)),
            pl.BlockSpec((96, 3), lambda b, j: (0, 0)),
            pl.BlockSpec((1, 3), lambda b, j: (0, 0)),
            pl.BlockSpec(memory_space=pl.ANY),
        ],
        out_specs=pl.BlockSpec((1, SC_CHUNK, 3), lambda b, j: (b, j, 0)),
        out_shape=jax.ShapeDtypeStruct((b_, n_, 3), jnp.float32),
        scratch_shapes=[
            pltpu.VMEM((MV_HW, 1, 32), jnp.float32),
            pltpu.VMEM((SC_CHUNK, 32), jnp.float32),
            pltpu.SemaphoreType.DMA,
        ],
        compiler_params=pltpu.CompilerParams(
            dimension_semantics=("parallel", "arbitrary")),
    )(*([rs(a) for a in mbil] + [f_half, f_half, fuse_w, fuse_b, mv_hwc]))


def _conv_x(x, w, b, stride=1):
    y = jax.lax.conv_general_dilated(x, w, (stride, stride), 'SAME',
                                     dimension_numbers=('NCHW', 'OIHW', 'NCHW'))
    return y + b[None, :, None, None]


def _up2x(x):
    return jnp.repeat(jnp.repeat(x, 2, axis=2), 2, axis=3)


def _smax(feat, rows, cols, bidx, nb, H, W):
    ri = jnp.clip(jnp.floor(rows).astype(jnp.int32), 0, H - 1)
    ci = jnp.clip(jnp.floor(cols).astype(jnp.int32), 0, W - 1)
    idx = (bidx * H + ri) * W + ci
    g = jax.ops.segment_max(feat, idx, num_segments=nb * H * W)
    g = jnp.where(jnp.isfinite(g), g, 0.0)
    return g.reshape(nb, H, W, -1).transpose(0, 3, 1, 2)


def _bilin(fmap, coords, scale):
    H, W = fmap.shape[2], fmap.shape[3]

    def one(f, rc):
        r = rc[:, 0] * scale
        c = rc[:, 1] * scale
        r0 = jnp.floor(r)
        c0 = jnp.floor(c)
        fr = r - r0
        fc = c - c0
        r0i = jnp.clip(r0.astype(jnp.int32), 0, H - 1)
        r1i = jnp.clip(r0i + 1, 0, H - 1)
        c0i = jnp.clip(c0.astype(jnp.int32), 0, W - 1)
        c1i = jnp.clip(c0i + 1, 0, W - 1)
        return (f[:, r0i, c0i] * (1 - fr) * (1 - fc)
                + f[:, r0i, c1i] * (1 - fr) * fc
                + f[:, r1i, c0i] * fr * (1 - fc)
                + f[:, r1i, c1i] * fr * fc)

    return jax.vmap(one)(fmap, coords)


def kernel(xyzi, des_coord, sph_coord, params):
    p = params
    b, t = B, T
    n = N

    x = xyzi[..., 0].reshape(b * t, 7, n).transpose(0, 2, 1)
    x8 = jnp.pad(x, ((0, 0), (0, 0), (0, 1)))
    w1 = jnp.pad(p['pn_w1'][:, :, 0, 0], ((0, 0), (0, 1))).T
    w2 = p['pn_w2'][:, :, 0, 0]
    f_half = _pointnet(x8, w1, p['pn_b1'][None, :], w2, p['pn_b2'][None, :])
    nchf = n // SC_CHUNK

    bev_rows = des_coord[:, :, :, 0, 0].reshape(-1)
    bev_cols = des_coord[:, :, :, 1, 0].reshape(-1)
    bev_idx = _linidx(bev_rows, bev_cols, BEV_H, BEV_W).reshape(b, t * n)
    bev_flat = _scatter_max_p(
        f_half, bev_idx, BEV_H * BEV_W,
        lambda bb, h, j: (t * bb + j // nchf, h, j % nchf, 0))
    bev_feat = (bev_flat.reshape(b, 2, BEV_H, BEV_W, 32)
                .transpose(0, 1, 4, 2, 3).reshape(b, PN_CH, BEV_H, BEV_W))

    rv_rows = sph_coord[:, -1, :, 1, 0].reshape(-1)
    rv_cols = sph_coord[:, -1, :, 0, 0].reshape(-1)
    rv_idx = _linidx(rv_rows, rv_cols, RV_H, RV_W).reshape(b, n)
    rv_flat = _scatter_max_p(
        f_half, rv_idx, RV_H * RV_W,
        lambda bb, h, j: (t * bb + (t - 1), h, j, 0))
    rv_feat = (rv_flat.reshape(b, 2, RV_H, RV_W, 32)
               .transpose(0, 1, 4, 2, 3).reshape(b, PN_CH, RV_H, RV_W))

    shallow = jax.nn.relu(_conv_x(rv_feat, p['rv_c1_w'], p['rv_c1_b'], 2))
    deep = jax.nn.relu(_conv_x(shallow, p['rv_c2_w'], p['rv_c2_b'], 2))
    movable_logit_2d = _up2x(_conv_x(shallow, p['rv_head_w'], p['rv_head_b']))

    rvr = sph_coord[:, -1, :, 1, 0].reshape(-1)
    rvc = sph_coord[:, -1, :, 0, 0].reshape(-1)
    sbil = _bilidx(rvr, rvc, RV_H // 2, RV_W // 2, 0.5)
    dbil = _bilidx(rvr, rvc, RV_H // 4, RV_W // 4, 0.25)

    bev_r0 = des_coord[:, -1, :, 0, 0]
    bev_c0 = des_coord[:, -1, :, 1, 0]
    sh_idx = _linidx((bev_r0 * 0.5).reshape(-1), (bev_c0 * 0.5).reshape(-1),
                     BEV_H // 2, BEV_W // 2).reshape(b, n)
    dp_idx = _linidx((bev_r0 * 0.25).reshape(-1), (bev_c0 * 0.25).reshape(-1),
                     BEV_H // 4, BEV_W // 4).reshape(b, n)

    sh_hwc = shallow.transpose(0, 2, 3, 1).reshape(b, RV_H // 2 * (RV_W // 2), 1, 32)
    dp_hwc = deep.transpose(0, 2, 3, 1).reshape(b, RV_H // 4 * (RV_W // 4), 1, 32)
    sh_flat, dp_flat = _rvproj(sh_hwc, dp_hwc, sh_idx, dp_idx, sbil, dbil)
    sh_bev = sh_flat.reshape(b, BEV_H // 2, BEV_W // 2, 32).transpose(0, 3, 1, 2)
    dp_bev = dp_flat.reshape(b, BEV_H // 4, BEV_W // 4, 32).transpose(0, 3, 1, 2)

    x1 = jax.nn.relu(_conv_x(bev_feat, p['bev_c1_w'], p['bev_c1_b'], 2))
    x1 = jax.nn.relu(_conv_x(jnp.concatenate([x1, sh_bev], 1), p['bev_c2_w'], p['bev_c2_b']))
    x2 = jax.nn.relu(_conv_x(x1, p['bev_c3_w'], p['bev_c3_b'], 2))
    x2 = jax.nn.relu(_conv_x(jnp.concatenate([x2, dp_bev], 1), p['bev_c4_w'], p['bev_c4_b']))
    moving_feat_2d = _conv_x(x1 + _up2x(x2), p['bev_c5_w'], p['bev_c5_b'])

    mbil = _bilidx(bev_r0.reshape(-1), bev_c0.reshape(-1),
                   BEV_H // 2, BEV_W // 2, 0.5)
    mv_hwc = moving_feat_2d.transpose(0, 2, 3, 1).reshape(b, MV_HW, 1, 32)
    logits = _fuse(f_half, mv_hwc, mbil, p['fuse_w'][:, :, 0, 0].T,
                   p['fuse_b'][None, :], t)
    moving_logit_3d = logits.transpose(0, 2, 1)[..., None]
    return moving_logit_3d, movable_logit_2d


# speedup vs baseline: 1.3287x; 1.0069x over previous
import functools

import jax
import jax.numpy as jnp
from jax.experimental import pallas as pl
from jax.experimental.pallas import tpu as pltpu

B, T, N = 2, 2, 131072
BEV_H = BEV_W = 512
RV_H, RV_W = 64, 2048
PN_CH = 64

PN_CHUNK = 4096
SC_CHUNK = 512
SC_U = 2

NEG = float("-inf")


def _pointnet_body(x_ref, w1_ref, b1_ref, w2_ref, b2_ref, out_ref):
    x = x_ref[0]
    valid = (x[:, 4:5] < 100.0).astype(jnp.float32)
    x = x * valid
    h = jnp.maximum(
        jax.lax.dot_general(x, w1_ref[...], (((1,), (0,)), ((), ())),
                            preferred_element_type=jnp.float32) + b1_ref[...],
        0.0)
    f = jnp.maximum(
        jax.lax.dot_general(h, w2_ref[...], (((1,), (1,)), ((), ())),
                            preferred_element_type=jnp.float32) + b2_ref[...],
        0.0)
    f = f * valid
    out_ref[0, 0] = f[:, :32]
    out_ref[0, 1] = f[:, 32:]


def _pointnet(x8, w1, b1, w2, b2):
    bt, n, _ = x8.shape
    grid = (bt, n // PN_CHUNK)
    return pl.pallas_call(
        _pointnet_body,
        grid=grid,
        in_specs=[
            pl.BlockSpec((1, PN_CHUNK, 8), lambda i, j: (i, j, 0)),
            pl.BlockSpec((8, 64), lambda i, j: (0, 0)),
            pl.BlockSpec((1, 64), lambda i, j: (0, 0)),
            pl.BlockSpec((64, 64), lambda i, j: (0, 0)),
            pl.BlockSpec((1, 64), lambda i, j: (0, 0)),
        ],
        out_specs=pl.BlockSpec((1, 2, PN_CHUNK, 32), lambda i, j: (i, 0, j, 0)),
        out_shape=jax.ShapeDtypeStruct((bt, 2, n, 32), jnp.float32),
        compiler_params=pltpu.CompilerParams(
            dimension_semantics=("parallel", "arbitrary")),
    )(x8, w1, b1, w2, b2)


def _rmw_batch(acc_ref, pts, cs, iota_cell):
    pk = 128 // cs
    pksh = {4: 2, 2: 1, 1: 0}[pk]
    bases, grps, masked = [], [], []
    for idx, fv in pts:
        row = idx >> pksh
        base = pl.multiple_of((row >> 3) << 3, 8)
        fv128 = jnp.concatenate([fv] * pk) if pk > 1 else fv
        m = jnp.where(iota_cell == (idx & (8 * pk - 1)), fv128[None, :], NEG)
        bases.append(base)
        grps.append(row >> 3)
        masked.append(m)
    nu = len(pts)
    loaded = [acc_ref[pl.ds(bases[k], 8), :] for k in range(nu)]
    vals = []
    for k in range(nu):
        t = loaded[k]
        for j2 in range(k):
            t = jnp.where(grps[j2] == grps[k], vals[j2], t)
        vals.append(jnp.maximum(t, masked[k]))
    for k in range(nu):
        acc_ref[pl.ds(bases[k], 8), :] = vals[k]


def _acc_finalize(acc_ref, out_at, sem, nrow):
    strip = 4096

    def _clean(r, _):
        o = acc_ref[pl.ds(r * strip, strip), :]
        acc_ref[pl.ds(r * strip, strip), :] = jnp.where(
            jnp.isfinite(o), o, 0.0)
        return 0

    jax.lax.fori_loop(0, max(nrow // strip, 1), _clean, 0)
    cp = pltpu.make_async_copy(acc_ref, out_at, sem)
    cp.start()
    cp.wait()


def _acc_init(acc_ref, nrow):
    strip = 4096

    def _fill(r, _):
        acc_ref[pl.ds(r * strip, strip), :] = jnp.full(
            (strip, 128), NEG, jnp.float32)
        return 0

    jax.lax.fori_loop(0, max(nrow // strip, 1), _fill, 0)


def _linidx_body(r_ref, c_ref, o_ref, *, H, W, rscale, cscale):
    ri = jnp.clip(jnp.floor(r_ref[...] * rscale).astype(jnp.int32), 0, H - 1)
    ci = jnp.clip(jnp.floor(c_ref[...] * cscale).astype(jnp.int32), 0, W - 1)
    o_ref[...] = ri * W + ci


def _linidx(rows, cols, H, W, rscale=1.0, cscale=1.0):
    m = rows.shape[0]
    r2 = rows.reshape(m // 1024, 1024)
    c2 = cols.reshape(m // 1024, 1024)
    out = pl.pallas_call(
        functools.partial(_linidx_body, H=H, W=W, rscale=rscale, cscale=cscale),
        grid=(m // 8192,),
        in_specs=[pl.BlockSpec((8, 1024), lambda i: (i, 0)),
                  pl.BlockSpec((8, 1024), lambda i: (i, 0))],
        out_specs=pl.BlockSpec((8, 1024), lambda i: (i, 0)),
        out_shape=jax.ShapeDtypeStruct((m // 1024, 1024), jnp.int32),
    )(r2, c2)
    return out.reshape(m)


def _scatter_body(idx_ref, feat_ref, out_ref, acc_ref, sem, *, hw, cs, nch):
    bi = pl.program_id(0)
    hi = pl.program_id(1)
    j = pl.program_id(2)
    nrow = hw // (128 // cs)

    @pl.when(j == 0)
    def _init():
        _acc_init(acc_ref, nrow)

    iota_cell = (jax.lax.broadcasted_iota(jnp.int32, (8, 128), 0) * (128 // cs)
                 + jax.lax.broadcasted_iota(jnp.int32, (8, 128), 1) // cs)
    for u0 in range(0, SC_CHUNK, SC_U):
        pts = [(idx_ref[0, 0, 0, u0 + k], feat_ref[0, 0, u0 + k, :])
               for k in range(SC_U)]
        _rmw_batch(acc_ref, pts, cs, iota_cell)

    @pl.when(j == nch - 1)
    def _fin():
        _acc_finalize(acc_ref, out_ref.at[bi, hi], sem, nrow)


def _scatter_max_p(feat, idx, hw, feat_map):
    g_, nh, n_, cs = feat.shape
    b_, m = idx.shape
    nch = m // SC_CHUNK
    idx4 = idx.reshape(b_, nch, 1, SC_CHUNK)
    body = functools.partial(_scatter_body, hw=hw, cs=cs, nch=nch)
    return pl.pallas_call(
        body,
        grid=(b_, nh, nch),
        in_specs=[
            pl.BlockSpec((1, 1, 1, SC_CHUNK), lambda b, h, j: (b, j, 0, 0),
                         memory_space=pltpu.SMEM),
            pl.BlockSpec((1, 1, SC_CHUNK, cs), feat_map),
        ],
        out_specs=pl.BlockSpec(memory_space=pl.ANY),
        out_shape=jax.ShapeDtypeStruct((b_, nh, hw // (128 // cs), 128),
                                       jnp.float32),
        scratch_shapes=[pltpu.VMEM((hw // (128 // cs), 128), jnp.float32),
                        pltpu.SemaphoreType.DMA],
        compiler_params=pltpu.CompilerParams(
            dimension_semantics=("parallel", "arbitrary", "arbitrary")),
    )(idx4, feat).reshape(b_, nh, hw, cs)


def _bilidx_body(r_ref, c_ref, i00_ref, dr_ref, dc_ref, fr_ref, fc_ref,
                 *, H, W, scale):
    r = r_ref[...] * scale
    c = c_ref[...] * scale
    rf = jnp.floor(r)
    cf = jnp.floor(c)
    fr_ref[...] = r - rf
    fc_ref[...] = c - cf
    r0 = jnp.clip(rf.astype(jnp.int32), 0, H - 1)
    r1 = jnp.clip(r0 + 1, 0, H - 1)
    c0 = jnp.clip(cf.astype(jnp.int32), 0, W - 1)
    c1 = jnp.clip(c0 + 1, 0, W - 1)
    i00_ref[...] = r0 * W + c0
    dr_ref[...] = (r1 - r0) * W
    dc_ref[...] = c1 - c0


def _bilidx(rows, cols, H, W, scale):
    m = rows.shape[0]
    r2 = rows.reshape(m // 1024, 1024)
    c2 = cols.reshape(m // 1024, 1024)
    sp = pl.BlockSpec((8, 1024), lambda i: (i, 0))
    outs = pl.pallas_call(
        functools.partial(_bilidx_body, H=H, W=W, scale=scale),
        grid=(m // 8192,),
        in_specs=[sp, sp],
        out_specs=[sp] * 5,
        out_shape=[jax.ShapeDtypeStruct((m // 1024, 1024), jnp.int32)] * 3
        + [jax.ShapeDtypeStruct((m // 1024, 1024), jnp.float32)] * 2,
    )(r2, c2)
    return [o.reshape(m) for o in outs]


def _gather_bilin(src_ref, i00, dr, dc, fr, fc):
    v00 = src_ref[i00, 0, :]
    v01 = src_ref[i00 + dc, 0, :]
    v10 = src_ref[i00 + dr, 0, :]
    v11 = src_ref[i00 + dr + dc, 0, :]
    top = v00 + fc * (v01 - v00)
    bot = v10 + fc * (v11 - v10)
    return top + fr * (bot - top)


SH_HW = 256 * 256
DP_HW = 128 * 128


def _rvproj_body(sidx_ref, didx_ref,
                 si_ref, sdr_ref, sdc_ref, sfr_ref, sfc_ref,
                 di_ref, ddr_ref, ddc_ref, dfr_ref, dfc_ref,
                 sh_hbm, dp_hbm, sh_out, dp_out,
                 sh_src, dp_src, acc_sh, acc_dp, sem1, sem2, *, nch):
    bi = pl.program_id(0)
    j = pl.program_id(1)

    @pl.when(j == 0)
    def _load():
        cp1 = pltpu.make_async_copy(sh_hbm.at[bi], sh_src, sem1)
        cp2 = pltpu.make_async_copy(dp_hbm.at[bi], dp_src, sem2)
        cp1.start()
        cp2.start()
        cp1.wait()
        cp2.wait()
        _acc_init(acc_sh, SH_HW // 4)
        _acc_init(acc_dp, DP_HW // 4)

    iota_cell = (jax.lax.broadcasted_iota(jnp.int32, (8, 128), 0) * 4
                 + jax.lax.broadcasted_iota(jnp.int32, (8, 128), 1) // 32)
    for u0 in range(0, SC_CHUNK, SC_U):
        sh_pts, dp_pts = [], []
        for k in range(SC_U):
            i = u0 + k
            sv = _gather_bilin(sh_src, si_ref[0, 0, 0, i], sdr_ref[0, 0, 0, i],
                               sdc_ref[0, 0, 0, i], sfr_ref[0, 0, 0, i],
                               sfc_ref[0, 0, 0, i])
            dv = _gather_bilin(dp_src, di_ref[0, 0, 0, i], ddr_ref[0, 0, 0, i],
                               ddc_ref[0, 0, 0, i], dfr_ref[0, 0, 0, i],
                               dfc_ref[0, 0, 0, i])
            sh_pts.append((sidx_ref[0, 0, 0, i], sv))
            dp_pts.append((didx_ref[0, 0, 0, i], dv))
        _rmw_batch(acc_sh, sh_pts, 32, iota_cell)
        _rmw_batch(acc_dp, dp_pts, 32, iota_cell)

    @pl.when(j == nch - 1)
    def _fin():
        _acc_finalize(acc_sh, sh_out.at[bi], sem1, SH_HW // 4)
        _acc_finalize(acc_dp, dp_out.at[bi], sem2, DP_HW // 4)


def _rvproj(shallow_hwc, deep_hwc, sidx, didx, sbil, dbil):
    b_ = shallow_hwc.shape[0]
    n_ = sidx.shape[1]
    nch = n_ // SC_CHUNK
    sm = lambda b, j: (b, j, 0, 0)
    smem = functools.partial(pl.BlockSpec, (1, 1, 1, SC_CHUNK),
                             memory_space=pltpu.SMEM)
    rs = lambda a: a.reshape(b_, nch, 1, SC_CHUNK)
    ins = ([rs(sidx), rs(didx)] + [rs(a) for a in sbil] + [rs(a) for a in dbil]
           + [shallow_hwc, deep_hwc])
    sh_hw4 = SH_HW // 4
    dp_hw4 = DP_HW // 4
    sh_out, dp_out = pl.pallas_call(
        functools.partial(_rvproj_body, nch=nch),
        grid=(b_, nch),
        in_specs=[smem(sm)] * 12 + [pl.BlockSpec(memory_space=pl.ANY)] * 2,
        out_specs=[pl.BlockSpec(memory_space=pl.ANY)] * 2,
        out_shape=[jax.ShapeDtypeStruct((b_, sh_hw4, 128), jnp.float32),
                   jax.ShapeDtypeStruct((b_, dp_hw4, 128), jnp.float32)],
        scratch_shapes=[
            pltpu.VMEM((32 * 1024, 1, 32), jnp.float32),
            pltpu.VMEM((16 * 512, 1, 32), jnp.float32),
            pltpu.VMEM((sh_hw4, 128), jnp.float32),
            pltpu.VMEM((dp_hw4, 128), jnp.float32),
            pltpu.SemaphoreType.DMA, pltpu.SemaphoreType.DMA,
        ],
        compiler_params=pltpu.CompilerParams(
            dimension_semantics=("parallel", "arbitrary")),
    )(*ins)
    return (sh_out.reshape(b_, 1, SH_HW, 32), dp_out.reshape(b_, 1, DP_HW, 32))


MV_HW = 256 * 256


def _fuse_body(mi_ref, mdr_ref, mdc_ref, mfr_ref, mfc_ref,
               f0a_ref, f0b_ref, w_ref, bias_ref, mv_hbm, out_ref,
               mv_src, gsl, sem, *, nch):
    bi = pl.program_id(0)
    j = pl.program_id(1)

    @pl.when(j == 0)
    def _load():
        cp = pltpu.make_async_copy(mv_hbm.at[bi], mv_src, sem)
        cp.start()
        cp.wait()

    for i in range(SC_CHUNK):
        gsl[i, :] = _gather_bilin(mv_src, mi_ref[0, 0, 0, i],
                                  mdr_ref[0, 0, 0, i], mdc_ref[0, 0, 0, i],
                                  mfr_ref[0, 0, 0, i], mfc_ref[0, 0, 0, i])
    dn = (((1,), (0,)), ((), ()))
    logits = (
        jax.lax.dot_general(f0a_ref[0, 0], w_ref[0:32], dn,
                            preferred_element_type=jnp.float32)
        + jax.lax.dot_general(f0b_ref[0, 0], w_ref[32:64], dn,
                              preferred_element_type=jnp.float32)
        + jax.lax.dot_general(gsl[...], w_ref[64:96], dn,
                              preferred_element_type=jnp.float32))
    out_ref[0] = logits + bias_ref[...]


def _fuse(f_half, mv_hwc, mbil, fuse_w, fuse_b, t):
    bt, _, n_, _ = f_half.shape
    b_ = bt // t
    nch = n_ // SC_CHUNK
    smem = functools.partial(pl.BlockSpec, (1, 1, 1, SC_CHUNK),
                             memory_space=pltpu.SMEM)
    sm = lambda b, j: (b, j, 0, 0)
    rs = lambda a: a.reshape(b_, nch, 1, SC_CHUNK)
    return pl.pallas_call(
        functools.partial(_fuse_body, nch=nch),
        grid=(b_, nch),
        in_specs=[smem(sm)] * 5 + [
            pl.BlockSpec((1, 1, SC_CHUNK, 32),
                         lambda b, j: (t * b + t - 1, 0, j, 0)),
            pl.BlockSpec((1, 1, SC_CHUNK, 32),
                         lambda b, j: (t * b + t - 1, 1, j, 0)),
            pl.BlockSpec((96, 3), lambda b, j: (0, 0)),
            pl.BlockSpec((1, 3), lambda b, j: (0, 0)),
            pl.BlockSpec(memory_space=pl.ANY),
        ],
        out_specs=pl.BlockSpec((1, SC_CHUNK, 3), lambda b, j: (b, j, 0)),
        out_shape=jax.ShapeDtypeStruct((b_, n_, 3), jnp.float32),
        scratch_shapes=[
            pltpu.VMEM((MV_HW, 1, 32), jnp.float32),
            pltpu.VMEM((SC_CHUNK, 32), jnp.float32),
            pltpu.SemaphoreType.DMA,
        ],
        compiler_params=pltpu.CompilerParams(
            dimension_semantics=("parallel", "arbitrary")),
    )(*([rs(a) for a in mbil] + [f_half, f_half, fuse_w, fuse_b, mv_hwc]))


def _conv_x(x, w, b, stride=1):
    y = jax.lax.conv_general_dilated(x, w, (stride, stride), 'SAME',
                                     dimension_numbers=('NCHW', 'OIHW', 'NCHW'))
    return y + b[None, :, None, None]


def _up2x(x):
    return jnp.repeat(jnp.repeat(x, 2, axis=2), 2, axis=3)


def _smax(feat, rows, cols, bidx, nb, H, W):
    ri = jnp.clip(jnp.floor(rows).astype(jnp.int32), 0, H - 1)
    ci = jnp.clip(jnp.floor(cols).astype(jnp.int32), 0, W - 1)
    idx = (bidx * H + ri) * W + ci
    g = jax.ops.segment_max(feat, idx, num_segments=nb * H * W)
    g = jnp.where(jnp.isfinite(g), g, 0.0)
    return g.reshape(nb, H, W, -1).transpose(0, 3, 1, 2)


def _bilin(fmap, coords, scale):
    H, W = fmap.shape[2], fmap.shape[3]

    def one(f, rc):
        r = rc[:, 0] * scale
        c = rc[:, 1] * scale
        r0 = jnp.floor(r)
        c0 = jnp.floor(c)
        fr = r - r0
        fc = c - c0
        r0i = jnp.clip(r0.astype(jnp.int32), 0, H - 1)
        r1i = jnp.clip(r0i + 1, 0, H - 1)
        c0i = jnp.clip(c0.astype(jnp.int32), 0, W - 1)
        c1i = jnp.clip(c0i + 1, 0, W - 1)
        return (f[:, r0i, c0i] * (1 - fr) * (1 - fc)
                + f[:, r0i, c1i] * (1 - fr) * fc
                + f[:, r1i, c0i] * fr * (1 - fc)
                + f[:, r1i, c1i] * fr * fc)

    return jax.vmap(one)(fmap, coords)


def kernel(xyzi, des_coord, sph_coord, params):
    p = params
    b, t = B, T
    n = N

    x = xyzi[..., 0].reshape(b * t, 7, n).transpose(0, 2, 1)
    x8 = jnp.pad(x, ((0, 0), (0, 0), (0, 1)))
    w1 = jnp.pad(p['pn_w1'][:, :, 0, 0], ((0, 0), (0, 1))).T
    w2 = p['pn_w2'][:, :, 0, 0]
    f_half = _pointnet(x8, w1, p['pn_b1'][None, :], w2, p['pn_b2'][None, :])
    nchf = n // SC_CHUNK

    bev_rows = des_coord[:, :, :, 0, 0].reshape(-1)
    bev_cols = des_coord[:, :, :, 1, 0].reshape(-1)
    bev_idx = _linidx(bev_rows, bev_cols, BEV_H, BEV_W).reshape(b, t * n)
    bev_flat = _scatter_max_p(
        f_half, bev_idx, BEV_H * BEV_W,
        lambda bb, h, j: (t * bb + j // nchf, h, j % nchf, 0))
    bev_feat = (bev_flat.reshape(b, 2, BEV_H, BEV_W, 32)
                .transpose(0, 1, 4, 2, 3).reshape(b, PN_CH, BEV_H, BEV_W))

    rv_rows = sph_coord[:, -1, :, 1, 0].reshape(-1)
    rv_cols = sph_coord[:, -1, :, 0, 0].reshape(-1)
    rv_idx = _linidx(rv_rows, rv_cols, RV_H, RV_W).reshape(b, n)
    rv_flat = _scatter_max_p(
        f_half, rv_idx, RV_H * RV_W,
        lambda bb, h, j: (t * bb + (t - 1), h, j, 0))
    rv_feat = (rv_flat.reshape(b, 2, RV_H, RV_W, 32)
               .transpose(0, 1, 4, 2, 3).reshape(b, PN_CH, RV_H, RV_W))

    shallow = jax.nn.relu(_conv_x(rv_feat, p['rv_c1_w'], p['rv_c1_b'], 2))
    deep = jax.nn.relu(_conv_x(shallow, p['rv_c2_w'], p['rv_c2_b'], 2))
    movable_logit_2d = _up2x(_conv_x(shallow, p['rv_head_w'], p['rv_head_b']))

    rvr = sph_coord[:, -1, :, 1, 0].reshape(-1)
    rvc = sph_coord[:, -1, :, 0, 0].reshape(-1)
    sbil = _bilidx(rvr, rvc, RV_H // 2, RV_W // 2, 0.5)
    dbil = _bilidx(rvr, rvc, RV_H // 4, RV_W // 4, 0.25)

    bev_r0 = des_coord[:, -1, :, 0, 0]
    bev_c0 = des_coord[:, -1, :, 1, 0]
    sh_idx = _linidx((bev_r0 * 0.5).reshape(-1), (bev_c0 * 0.5).reshape(-1),
                     BEV_H // 2, BEV_W // 2).reshape(b, n)
    dp_idx = _linidx((bev_r0 * 0.25).reshape(-1), (bev_c0 * 0.25).reshape(-1),
                     BEV_H // 4, BEV_W // 4).reshape(b, n)

    sh_hwc = shallow.transpose(0, 2, 3, 1).reshape(b, RV_H // 2 * (RV_W // 2), 1, 32)
    dp_hwc = deep.transpose(0, 2, 3, 1).reshape(b, RV_H // 4 * (RV_W // 4), 1, 32)
    sh_flat, dp_flat = _rvproj(sh_hwc, dp_hwc, sh_idx, dp_idx, sbil, dbil)
    sh_bev = sh_flat.reshape(b, BEV_H // 2, BEV_W // 2, 32).transpose(0, 3, 1, 2)
    dp_bev = dp_flat.reshape(b, BEV_H // 4, BEV_W // 4, 32).transpose(0, 3, 1, 2)

    x1 = jax.nn.relu(_conv_x(bev_feat, p['bev_c1_w'], p['bev_c1_b'], 2))
    x1 = jax.nn.relu(_conv_x(jnp.concatenate([x1, sh_bev], 1), p['bev_c2_w'], p['bev_c2_b']))
    x2 = jax.nn.relu(_conv_x(x1, p['bev_c3_w'], p['bev_c3_b'], 2))
    x2 = jax.nn.relu(_conv_x(jnp.concatenate([x2, dp_bev], 1), p['bev_c4_w'], p['bev_c4_b']))
    moving_feat_2d = _conv_x(x1 + _up2x(x2), p['bev_c5_w'], p['bev_c5_b'])

    mbil = _bilidx(bev_r0.reshape(-1), bev_c0.reshape(-1),
                   BEV_H // 2, BEV_W // 2, 0.5)
    mv_hwc = moving_feat_2d.transpose(0, 2, 3, 1).reshape(b, MV_HW, 1, 32)
    logits = _fuse(f_half, mv_hwc, mbil, p['fuse_w'][:, :, 0, 0].T,
                   p['fuse_b'][None, :], t)
    moving_logit_3d = logits.transpose(0, 2, 1)[..., None]
    return moving_logit_3d, movable_logit_2d
